# baseline (speedup 1.0000x reference)
import functools

import jax
import jax.numpy as jnp
from jax import lax
from jax.experimental import pallas as pl
from jax.experimental.pallas import tpu as pltpu
from jax.experimental.pallas import tpu_sc as plsc

N_ATOMS = 25000
N_EDGES = 50000
N_MOLS = 1250
D_V = 72
D_E = 14
D_H = 2048
DEPTH = 3

NA_P = 25088
NE_P = 51200
NM_P = 1280

NC = 2
NS = 16

CW = 128
NCB = D_H // CW
ECH = 128
E_PER_TILE = NE_P // NS
NECH = E_PER_TILE // ECH
ACH = 112
A_PER_TILE = NA_P // NS
NACH = A_PER_TILE // ACH
M_PER_TILE = NM_P // NS
ZROWS = 72
ZROWS_M = 80

HALF = NA_P // 2
ACC_R = 12672
JR = HALF
ZPT = ACC_R // NS
W_PT = HALF // NS

GCH = 64
G_PER_W = NE_P // (NC * NS)
NGCH = G_PER_W // GCH


@functools.cache
def _sc_mesh():
    return plsc.VectorSubcoreMesh(
        core_axis_name="c", subcore_axis_name="s", num_cores=NC, num_subcores=NS)


def _sc_gather_rows(table, idx):
    b, = idx.shape
    _, w = table.shape
    nw = NC * NS
    chunk = 80
    nch = b // (nw * chunk)
    assert b == nw * chunk * nch

    @functools.partial(
        pl.kernel,
        out_type=jax.ShapeDtypeStruct((b, w), jnp.float32),
        mesh=_sc_mesh(),
        scratch_types=[
            pltpu.VMEM((chunk,), jnp.int32),
            pltpu.VMEM((chunk, w), jnp.float32),
            pltpu.SemaphoreType.DMA,
        ],
    )
    def k(table_hbm, idx_hbm, out_hbm, idx_v, rows_v, sem):
        wid = lax.axis_index("s") * NC + lax.axis_index("c")
        base = wid * chunk * nch

        def body(j, _):
            e0 = pl.multiple_of(base + j * chunk, 8)
            pltpu.sync_copy(idx_hbm.at[pl.ds(e0, chunk)], idx_v)
            pltpu.async_copy(table_hbm.at[idx_v], rows_v, sem).wait()
            pltpu.sync_copy(rows_v, out_hbm.at[pl.ds(e0, chunk)])
            return 0

        lax.fori_loop(0, nch, body, 0)

    return k(table, idx)


def _sc_segsum(H, dst3, z0, out16):
    if out16:
        out_type = tuple(jax.ShapeDtypeStruct((NA_P, CW), jnp.float32)
                         for _ in range(NCB))
    else:
        out_type = jax.ShapeDtypeStruct((NA_P, D_H), jnp.float32)

    @functools.partial(
        pl.kernel,
        out_type=out_type,
        mesh=_sc_mesh(),
        scratch_types=[
            pltpu.VMEM_SHARED((ACC_R, CW), jnp.float32),
            pltpu.VMEM((ZROWS, CW), jnp.float32),
            pltpu.VMEM((ECH, CW), jnp.float32),
            pltpu.VMEM((NECH, ECH), jnp.int32),
        ],
    )
    def k(h_hbm, dst_hbm, z0_hbm, *rest):
        outs = rest[:NCB] if out16 else rest[:1]
        acc, ztile, rows, idxd = rest[NCB if out16 else 1:]
        cid = lax.axis_index("c")
        sid = lax.axis_index("s")
        pltpu.sync_copy(z0_hbm.at[pl.ds(0, ZROWS)], ztile)
        pltpu.sync_copy(dst_hbm.at[sid], idxd)
        lo = cid * HALF

        def tloop(r, _):
            for v in range(ECH // 16):
                x = idxd[r, pl.ds(v * 16, 16)]
                shifted = x - lo
                ok = (shifted >= 0) & (shifted < HALF)
                idxd[r, pl.ds(v * 16, 16)] = jnp.where(ok, shifted, JR)
            return 0

        lax.fori_loop(0, NECH, tloop, 0)

        for cb in range(NCB):
            c0 = cb * CW
            for i in range(ZPT // ZROWS):
                r0 = pl.multiple_of(sid * ZPT + i * ZROWS, 8)
                pltpu.sync_copy(ztile, acc.at[pl.ds(r0, ZROWS)])
            plsc.subcore_barrier()

            def sloop(j, _):
                e0 = pl.multiple_of(sid * E_PER_TILE + j * ECH, 8)
                pltpu.sync_copy(h_hbm.at[pl.ds(e0, ECH), pl.ds(c0, CW)], rows)
                pltpu.sync_copy(rows, acc.at[idxd.at[j]], add=True)
                return 0

            lax.fori_loop(0, NECH, sloop, 0)
            plsc.subcore_barrier()
            a0 = pl.multiple_of(sid * W_PT, 8)
            g0 = pl.multiple_of(lo + sid * W_PT, 8)
            if out16:
                pltpu.sync_copy(acc.at[pl.ds(a0, W_PT)],
                                outs[cb].at[pl.ds(g0, W_PT)])
            else:
                pltpu.sync_copy(acc.at[pl.ds(a0, W_PT)],
                                outs[0].at[pl.ds(g0, W_PT), pl.ds(c0, CW)])
            plsc.subcore_barrier()

    return k(H, dst3, z0)


def _sc_gather_matom(mats, src):
    @functools.partial(
        pl.kernel,
        out_type=jax.ShapeDtypeStruct((NE_P, D_H), jnp.float32),
        mesh=_sc_mesh(),
        scratch_types=[
            pltpu.VMEM((GCH,), jnp.int32),
            pltpu.VMEM((GCH, CW), jnp.float32),
            pltpu.SemaphoreType.DMA,
        ],
    )
    def k(*refs):
        mat_refs = refs[:NCB]
        src_hbm = refs[NCB]
        g_hbm = refs[NCB + 1]
        idx_v, rows_v, sem = refs[NCB + 2:]
        wid = lax.axis_index("s") * NC + lax.axis_index("c")
        base = wid * G_PER_W

        def body(j, _):
            e0 = pl.multiple_of(base + j * GCH, 8)
            pltpu.sync_copy(src_hbm.at[pl.ds(e0, GCH)], idx_v)
            for cb in range(NCB):
                pltpu.async_copy(mat_refs[cb].at[idx_v], rows_v, sem).wait()
                pltpu.sync_copy(rows_v,
                                g_hbm.at[pl.ds(e0, GCH), pl.ds(cb * CW, CW)])
            return 0

        lax.fori_loop(0, NGCH, body, 0)

    return k(*mats, src)


def _sc_mol_sum(Hv, batch3, ones_b, z0):
    @functools.partial(
        pl.kernel,
        out_type=(
            jax.ShapeDtypeStruct((NM_P, D_H), jnp.float32),
            jax.ShapeDtypeStruct((NM_P, CW), jnp.float32),
        ),
        mesh=_sc_mesh(),
        scratch_types=[
            pltpu.VMEM_SHARED((NM_P, CW), jnp.float32),
            pltpu.VMEM_SHARED((NM_P, CW), jnp.float32),
            pltpu.VMEM((ZROWS_M, CW), jnp.float32),
            pltpu.VMEM((ACH, CW), jnp.float32),
            pltpu.VMEM((ACH, CW), jnp.float32),
            pltpu.VMEM((NACH, ACH), jnp.int32),
        ],
    )
    def k(hv_hbm, b_hbm, ones_hbm, z0_hbm, zs_hbm, cnt_hbm,
          accz, accc, ztile, rows, onesb, idxb):
        cid = lax.axis_index("c")
        sid = lax.axis_index("s")
        pltpu.sync_copy(z0_hbm, ztile)
        pltpu.sync_copy(ones_hbm, onesb)
        pltpu.sync_copy(b_hbm.at[sid], idxb)
        r0 = pl.multiple_of(sid * M_PER_TILE, 8)

        pltpu.sync_copy(ztile.at[pl.ds(0, M_PER_TILE)],
                        accc.at[pl.ds(r0, M_PER_TILE)])
        plsc.subcore_barrier()

        def cloop(j, _):
            pltpu.sync_copy(onesb, accc.at[idxb.at[j]], add=True)
            return 0

        lax.fori_loop(0, NACH, cloop, 0)
        plsc.subcore_barrier()
        pltpu.sync_copy(accc.at[pl.ds(r0, M_PER_TILE)],
                        cnt_hbm.at[pl.ds(r0, M_PER_TILE)])

        for cb in range(NCB // NC):
            c0 = pl.multiple_of((cb * NC + cid) * CW, CW)
            pltpu.sync_copy(ztile.at[pl.ds(0, M_PER_TILE)],
                            accz.at[pl.ds(r0, M_PER_TILE)])
            plsc.subcore_barrier()

            def sloop(j, _):
                a0 = pl.multiple_of(sid * A_PER_TILE + j * ACH, 8)
                pltpu.sync_copy(hv_hbm.at[pl.ds(a0, ACH), pl.ds(c0, CW)], rows)
                pltpu.sync_copy(rows, accz.at[idxb.at[j]], add=True)
                return 0

            lax.fori_loop(0, NACH, sloop, 0)
            plsc.subcore_barrier()
            pltpu.sync_copy(accz.at[pl.ds(r0, M_PER_TILE)],
                            zs_hbm.at[pl.ds(r0, M_PER_TILE), pl.ds(c0, CW)])
            plsc.subcore_barrier()

    return k(Hv, batch3, ones_b, z0)


def _tc_init(Vg, Ep, W1v, W1e):
    BM = 512

    def body(vg_ref, e_ref, wv_ref, we_ref, h0_ref, h_ref):
        acc = jnp.dot(vg_ref[...], wv_ref[...], preferred_element_type=jnp.float32)
        acc = acc + jnp.dot(e_ref[...], we_ref[...], preferred_element_type=jnp.float32)
        h0_ref[...] = acc
        h_ref[...] = jnp.maximum(acc, 0.0)

    return pl.pallas_call(
        body,
        grid=(NE_P // BM,),
        in_specs=[
            pl.BlockSpec((BM, 128), lambda i: (i, 0)),
            pl.BlockSpec((BM, 16), lambda i: (i, 0)),
            pl.BlockSpec((128, D_H), lambda i: (0, 0)),
            pl.BlockSpec((16, D_H), lambda i: (0, 0)),
        ],
        out_specs=[
            pl.BlockSpec((BM, D_H), lambda i: (i, 0)),
            pl.BlockSpec((BM, D_H), lambda i: (i, 0)),
        ],
        out_shape=[
            jax.ShapeDtypeStruct((NE_P, D_H), jnp.float32),
            jax.ShapeDtypeStruct((NE_P, D_H), jnp.float32),
        ],
        compiler_params=pltpu.CompilerParams(
            dimension_semantics=("parallel",),
        ),
    )(Vg, Ep, W1v, W1e)


def _pair_swap(x, bm, bk):
    rows = lax.broadcasted_iota(jnp.int32, (bm, bk), 0)
    dn = pltpu.roll(x, bm - 1, 0)
    up = pltpu.roll(x, 1, 0)
    return jnp.where(rows % 2 == 0, dn, up)


def _tc_update(G, H, H0, W_h):
    BM = 512
    BK = 512
    KG = D_H // BK

    def body(g_ref, h_ref, h0_ref, w_ref, out_ref):
        k = pl.program_id(1)
        m = g_ref[...] - _pair_swap(h_ref[...], BM, BK)
        acc = jnp.dot(m, w_ref[...], preferred_element_type=jnp.float32)

        @pl.when(k == 0)
        def _():
            out_ref[...] = acc

        @pl.when(k > 0)
        def _():
            out_ref[...] = out_ref[...] + acc

        @pl.when(k == KG - 1)
        def _():
            out_ref[...] = jnp.maximum(out_ref[...] + h0_ref[...], 0.0)

    return pl.pallas_call(
        body,
        grid=(NE_P // BM, KG),
        in_specs=[
            pl.BlockSpec((BM, BK), lambda i, k: (i, k)),
            pl.BlockSpec((BM, BK), lambda i, k: (i, k)),
            pl.BlockSpec((BM, D_H), lambda i, k: (i, 0)),
            pl.BlockSpec((BK, D_H), lambda i, k: (k, 0)),
        ],
        out_specs=pl.BlockSpec((BM, D_H), lambda i, k: (i, 0)),
        out_shape=jax.ShapeDtypeStruct((NE_P, D_H), jnp.float32),
        compiler_params=pltpu.CompilerParams(
            dimension_semantics=("parallel", "arbitrary"),
        ),
    )(G, H, H0, W_h)


def _tc_final(Vp, Mv, Wov, Wom, bo):
    BM = 512
    BK = 512
    KG = D_H // BK

    def body(v_ref, mv_ref, wv_ref, wm_ref, b_ref, out_ref):
        k = pl.program_id(1)
        acc = jnp.dot(mv_ref[...], wm_ref[...], preferred_element_type=jnp.float32)

        @pl.when(k == 0)
        def _():
            out_ref[...] = acc + jnp.dot(v_ref[...], wv_ref[...],
                                         preferred_element_type=jnp.float32)

        @pl.when(k > 0)
        def _():
            out_ref[...] = out_ref[...] + acc

        @pl.when(k == KG - 1)
        def _():
            out_ref[...] = jnp.maximum(out_ref[...] + b_ref[0:1, :], 0.0)

    return pl.pallas_call(
        body,
        grid=(NA_P // BM, KG),
        in_specs=[
            pl.BlockSpec((BM, 128), lambda i, k: (i, 0)),
            pl.BlockSpec((BM, BK), lambda i, k: (i, k)),
            pl.BlockSpec((128, D_H), lambda i, k: (0, 0)),
            pl.BlockSpec((BK, D_H), lambda i, k: (k, 0)),
            pl.BlockSpec((8, D_H), lambda i, k: (0, 0)),
        ],
        out_specs=pl.BlockSpec((BM, D_H), lambda i, k: (i, 0)),
        out_shape=jax.ShapeDtypeStruct((NA_P, D_H), jnp.float32),
        compiler_params=pltpu.CompilerParams(
            dimension_semantics=("parallel", "arbitrary"),
        ),
    )(Vp, Mv, Wov, Wom, bo)


def _tc_divide(Zs, cnt):
    BM = 256

    def body(zs_ref, c_ref, out_ref):
        c = jnp.maximum(c_ref[:, 0:1], 1.0)
        out_ref[...] = zs_ref[...] / c

    return pl.pallas_call(
        body,
        grid=(NM_P // BM,),
        in_specs=[
            pl.BlockSpec((BM, D_H), lambda i: (i, 0)),
            pl.BlockSpec((BM, CW), lambda i: (i, 0)),
        ],
        out_specs=pl.BlockSpec((BM, D_H), lambda i: (i, 0)),
        out_shape=jax.ShapeDtypeStruct((NM_P, D_H), jnp.float32),
        compiler_params=pltpu.CompilerParams(
            dimension_semantics=("parallel",),
        ),
    )(Zs, cnt)


def kernel(V, E, edge_index, rev_edge_index, batch, W_i, W_h, W_o, b_o):
    del rev_edge_index
    src = edge_index[0]
    dst = edge_index[1]

    Vp = jnp.pad(V, ((0, NA_P - N_ATOMS), (0, 128 - D_V)))
    Ep = jnp.pad(E, ((0, NE_P - N_EDGES), (0, 16 - D_E)))
    src_p = jnp.pad(src, (0, NE_P - N_EDGES))
    dst_p = jnp.pad(dst, (0, NE_P - N_EDGES),
                    constant_values=NA_P - 1)
    batch_p = jnp.pad(batch, (0, NA_P - N_ATOMS),
                      constant_values=NM_P - 1)
    dst3 = dst_p.reshape(NS, NECH, ECH)
    batch3 = batch_p.reshape(NS, NACH, ACH)
    z0 = jnp.zeros((ZROWS_M, CW), jnp.float32)
    ones_b = jnp.ones((ACH, CW), jnp.float32)

    W1v = jnp.pad(W_i[:D_V], ((0, 128 - D_V), (0, 0)))
    W1e = jnp.pad(W_i[D_V:], ((0, 16 - D_E), (0, 0)))
    Wov = jnp.pad(W_o[:D_V], ((0, 128 - D_V), (0, 0)))
    Wom = W_o[D_V:]
    bo2 = jnp.broadcast_to(b_o[None, :], (8, D_H))

    Vg = _sc_gather_rows(Vp, src_p)
    H0, H = _tc_init(Vg, Ep, W1v, W1e)
    for _ in range(1, DEPTH):
        mats = _sc_segsum(H, dst3, z0, out16=True)
        G = _sc_gather_matom(mats, src_p)
        H = _tc_update(G, H, H0, W_h)
    Mv = _sc_segsum(H, dst3, z0, out16=False)
    Hv = _tc_final(Vp, Mv, Wov, Wom, bo2)
    Zs, cnt = _sc_mol_sum(Hv, batch3, ones_b, z0)
    Z = _tc_divide(Zs, cnt)
    return Z[:N_MOLS]


# speedup vs baseline: 1.5047x; 1.5047x over previous
import functools

import jax
import jax.numpy as jnp
from jax import lax
from jax.experimental import pallas as pl
from jax.experimental.pallas import tpu as pltpu
from jax.experimental.pallas import tpu_sc as plsc

N_ATOMS = 25000
N_EDGES = 50000
N_MOLS = 1250
D_V = 72
D_E = 14
D_H = 2048
DEPTH = 3

NA_P = 25088
NE_P = 51200
NM_P = 1280

NC = 2
NS = 16

CW = 128
NCB = D_H // CW
ECH = 64
E_PER_TILE = NE_P // NS
NECH = E_PER_TILE // ECH
ACH = 112
A_PER_TILE = NA_P // NS
NACH = A_PER_TILE // ACH
M_PER_TILE = NM_P // NS
ZROWS = 24
ZROWS_M = 80

HALF = NA_P // 2
ACC_R = 12672
JR = HALF // NS
ZPT = ACC_R // NS
W_PT = HALF // NS

G_PER_W = NE_P // (NC * NS)
GCH = 64
NGCH = G_PER_W // GCH


@functools.cache
def _sc_mesh():
    return plsc.VectorSubcoreMesh(
        core_axis_name="c", subcore_axis_name="s", num_cores=NC, num_subcores=NS)


def _sc_gather_rows(table, idx3):
    nw, nch, chunk = idx3.shape
    b = nw * nch * chunk
    _, w = table.shape
    assert nw == NC * NS and nch % 4 == 0

    @functools.partial(
        pl.kernel,
        out_type=jax.ShapeDtypeStruct((b, w), jnp.float32),
        mesh=_sc_mesh(),
        scratch_types=[
            pltpu.VMEM((nch, chunk), jnp.int32),
            pltpu.VMEM((chunk, w), jnp.float32),
            pltpu.VMEM((chunk, w), jnp.float32),
            pltpu.VMEM((chunk, w), jnp.float32),
            pltpu.VMEM((chunk, w), jnp.float32),
            pltpu.SemaphoreType.DMA,
            pltpu.SemaphoreType.DMA,
            pltpu.SemaphoreType.DMA,
            pltpu.SemaphoreType.DMA,
            pltpu.SemaphoreType.DMA,
            pltpu.SemaphoreType.DMA,
            pltpu.SemaphoreType.DMA,
            pltpu.SemaphoreType.DMA,
        ],
    )
    def k(table_hbm, idx_hbm, out_hbm, idx_v, r0_, r1_, r2_, r3_, *sems):
        rows = (r0_, r1_, r2_, r3_)
        gsem = sems[:4]
        wsem = sems[4:]
        wid = lax.axis_index("s") * NC + lax.axis_index("c")
        base = wid * chunk * nch
        pltpu.sync_copy(idx_hbm.at[wid], idx_v)

        def body(t, _):
            dg = []
            dw = []
            for i in range(4):
                q = 4 * t + i
                dg.append(pltpu.async_copy(table_hbm.at[idx_v.at[q]],
                                           rows[i], gsem[i]))
            for i in range(4):
                q = 4 * t + i
                e0 = pl.multiple_of(base + q * chunk, 8)
                dg[i].wait()
                dw.append(pltpu.async_copy(rows[i],
                                           out_hbm.at[pl.ds(e0, chunk)],
                                           wsem[i]))
            for d in dw:
                d.wait()
            return 0

        lax.fori_loop(0, nch // 4, body, 0)

    return k(table, idx3)


def _sc_segsum(H, dst3, z0, out16):
    if out16:
        out_type = tuple(jax.ShapeDtypeStruct((NA_P, CW), jnp.float32)
                         for _ in range(NCB))
    else:
        out_type = jax.ShapeDtypeStruct((NA_P, D_H), jnp.float32)

    @functools.partial(
        pl.kernel,
        out_type=out_type,
        mesh=_sc_mesh(),
        scratch_types=[
            pltpu.VMEM_SHARED((ACC_R, CW), jnp.float32),
            pltpu.VMEM((ZROWS, CW), jnp.float32),
            pltpu.VMEM((ECH, CW), jnp.float32),
            pltpu.VMEM((ECH, CW), jnp.float32),
            pltpu.VMEM((NECH, ECH), jnp.int32),
            pltpu.SemaphoreType.DMA,
            pltpu.SemaphoreType.DMA,
            pltpu.SemaphoreType.DMA,
            pltpu.SemaphoreType.DMA,
            pltpu.SemaphoreType.DMA,
        ],
    )
    def k(h_hbm, dst_hbm, z0_hbm, *rest):
        outs = rest[:NCB] if out16 else rest[:1]
        (acc, ztile, rows0, rows1, idxd,
         s0, s1, ss0, ss1, zsem) = rest[NCB if out16 else 1:]
        cid = lax.axis_index("c")
        sid = lax.axis_index("s")
        pltpu.sync_copy(z0_hbm.at[pl.ds(0, ZROWS)], ztile)
        pltpu.sync_copy(dst_hbm.at[sid], idxd)
        lo = cid * HALF

        def tloop(r, _):
            for v in range(ECH // 16):
                x = idxd[r, pl.ds(v * 16, 16)]
                shifted = x - lo
                ok = (shifted >= 0) & (shifted < HALF)
                tile = jnp.right_shift(
                    jnp.right_shift(shifted, 4) * 42800, 21)
                idxd[r, pl.ds(v * 16, 16)] = jnp.where(
                    ok, shifted + tile * 8, JR)
            return 0

        lax.fori_loop(0, NECH, tloop, 0)

        def h_src(q, c0):
            e0 = pl.multiple_of(sid * E_PER_TILE + q * ECH, 8)
            return h_hbm.at[pl.ds(e0, ECH), pl.ds(c0, CW)]

        for cb in range(NCB):
            c0 = cb * CW
            zd = [pltpu.async_copy(
                      ztile,
                      acc.at[pl.ds(pl.multiple_of(sid * ZPT + i * ZROWS, 8),
                                   ZROWS)],
                      zsem)
                  for i in range(ZPT // ZROWS)]
            for d in zd:
                d.wait()
            plsc.subcore_barrier()

            pltpu.async_copy(h_src(0, c0), rows0, s0)
            pltpu.async_copy(h_src(1, c0), rows1, s1)

            def sloop(j, _):
                a = 2 * j
                b = 2 * j + 1
                na = jnp.where(a + 2 < NECH, a + 2, 0)
                nb = jnp.where(b + 2 < NECH, b + 2, 1)
                pltpu.make_async_copy(h_src(a, c0), rows0, s0).wait()
                da = pltpu.async_copy(rows0, acc.at[idxd.at[a]], ss0, add=True)
                pltpu.make_async_copy(h_src(b, c0), rows1, s1).wait()
                db = pltpu.async_copy(rows1, acc.at[idxd.at[b]], ss1, add=True)
                da.wait()
                pltpu.async_copy(h_src(na, c0), rows0, s0)
                db.wait()
                pltpu.async_copy(h_src(nb, c0), rows1, s1)
                return 0

            lax.fori_loop(0, NECH // 2, sloop, 0)
            pltpu.make_async_copy(h_src(0, c0), rows0, s0).wait()
            pltpu.make_async_copy(h_src(1, c0), rows1, s1).wait()
            plsc.subcore_barrier()
            a0 = pl.multiple_of(sid * ZPT, 8)
            g0 = pl.multiple_of(lo + sid * W_PT, 8)
            if out16:
                pltpu.sync_copy(acc.at[pl.ds(a0, W_PT)],
                                outs[cb].at[pl.ds(g0, W_PT)])
            else:
                pltpu.sync_copy(acc.at[pl.ds(a0, W_PT)],
                                outs[0].at[pl.ds(g0, W_PT), pl.ds(c0, CW)])

    return k(H, dst3, z0)


def _sc_gather_matom(mats, src3):
    @functools.partial(
        pl.kernel,
        out_type=jax.ShapeDtypeStruct((NE_P, D_H), jnp.float32),
        mesh=_sc_mesh(),
        scratch_types=[
            pltpu.VMEM((NGCH, GCH), jnp.int32),
            pltpu.VMEM((GCH, CW), jnp.float32),
            pltpu.VMEM((GCH, CW), jnp.float32),
            pltpu.VMEM((GCH, CW), jnp.float32),
            pltpu.VMEM((GCH, CW), jnp.float32),
            pltpu.SemaphoreType.DMA,
            pltpu.SemaphoreType.DMA,
            pltpu.SemaphoreType.DMA,
            pltpu.SemaphoreType.DMA,
            pltpu.SemaphoreType.DMA,
            pltpu.SemaphoreType.DMA,
            pltpu.SemaphoreType.DMA,
            pltpu.SemaphoreType.DMA,
        ],
    )
    def k(*refs):
        mat_refs = refs[:NCB]
        src_hbm = refs[NCB]
        g_hbm = refs[NCB + 1]
        idx_a = refs[NCB + 2]
        rows = refs[NCB + 3:NCB + 7]
        gsem = refs[NCB + 7:NCB + 11]
        wsem = refs[NCB + 11:NCB + 15]
        wid = lax.axis_index("s") * NC + lax.axis_index("c")
        base = wid * G_PER_W
        pltpu.sync_copy(src_hbm.at[wid], idx_a)

        def g_dst(q, cb):
            e0 = pl.multiple_of(base + q * GCH, 8)
            return g_hbm.at[pl.ds(e0, GCH), pl.ds(cb * CW, CW)]

        def body(q, _):
            dg = {}
            dw = {}
            for cb in range(NCB):
                b = cb % 4
                if cb >= 4:
                    dw.pop(cb - 4).wait()
                dg[cb] = pltpu.async_copy(mat_refs[cb].at[idx_a.at[q]],
                                          rows[b], gsem[b])
                if cb >= 2:
                    u = cb - 2
                    dg.pop(u).wait()
                    dw[u] = pltpu.async_copy(rows[u % 4], g_dst(q, u),
                                             wsem[u % 4])
            for u in (NCB - 2, NCB - 1):
                dg.pop(u).wait()
                dw[u] = pltpu.async_copy(rows[u % 4], g_dst(q, u),
                                         wsem[u % 4])
            for u in range(NCB - 4, NCB):
                dw.pop(u).wait()
            return 0

        lax.fori_loop(0, NGCH, body, 0)

    return k(*mats, src3)


def _sc_mol_mean(Hv, batch3, ones_b, z0):
    @functools.partial(
        pl.kernel,
        out_type=jax.ShapeDtypeStruct((NM_P, D_H), jnp.float32),
        mesh=_sc_mesh(),
        scratch_types=[
            pltpu.VMEM_SHARED((NM_P, CW), jnp.float32),
            pltpu.VMEM_SHARED((NM_P, CW), jnp.float32),
            pltpu.VMEM((ZROWS_M, CW), jnp.float32),
            pltpu.VMEM((ACH, CW), jnp.float32),
            pltpu.VMEM((ACH, CW), jnp.float32),
            pltpu.VMEM((ACH, CW), jnp.float32),
            pltpu.VMEM((M_PER_TILE, CW), jnp.float32),
            pltpu.VMEM((M_PER_TILE, CW), jnp.float32),
            pltpu.VMEM((NACH, ACH), jnp.int32),
            pltpu.SemaphoreType.DMA,
            pltpu.SemaphoreType.DMA,
        ],
    )
    def k(hv_hbm, b_hbm, ones_hbm, z0_hbm, z_hbm,
          accz, accc, ztile, rows0, rows1, onesb, zrows, crows, idxb, s0, s1):
        cid = lax.axis_index("c")
        sid = lax.axis_index("s")
        pltpu.sync_copy(z0_hbm, ztile)
        pltpu.sync_copy(ones_hbm, onesb)
        pltpu.sync_copy(b_hbm.at[sid], idxb)
        r0 = pl.multiple_of(sid * M_PER_TILE, 8)

        pltpu.sync_copy(ztile.at[pl.ds(0, M_PER_TILE)],
                        accc.at[pl.ds(r0, M_PER_TILE)])
        plsc.subcore_barrier()

        def cloop(j, _):
            pltpu.sync_copy(onesb, accc.at[idxb.at[j]], add=True)
            return 0

        lax.fori_loop(0, NACH, cloop, 0)
        plsc.subcore_barrier()
        pltpu.sync_copy(accc.at[pl.ds(r0, M_PER_TILE)], crows)

        def hv_src(q, c0):
            a0 = pl.multiple_of(sid * A_PER_TILE + q * ACH, 8)
            return hv_hbm.at[pl.ds(a0, ACH), pl.ds(c0, CW)]

        for cb in range(NCB // NC):
            c0 = pl.multiple_of((cb * NC + cid) * CW, CW)
            pltpu.sync_copy(ztile.at[pl.ds(0, M_PER_TILE)],
                            accz.at[pl.ds(r0, M_PER_TILE)])
            plsc.subcore_barrier()

            pltpu.async_copy(hv_src(0, c0), rows0, s0)

            def sloop(j, _):
                a = 2 * j
                b = 2 * j + 1
                nxt = jnp.where(b + 1 < NACH, b + 1, 0)
                pltpu.make_async_copy(hv_src(a, c0), rows0, s0).wait()
                pltpu.async_copy(hv_src(b, c0), rows1, s1)
                pltpu.sync_copy(rows0, accz.at[idxb.at[a]], add=True)
                pltpu.make_async_copy(hv_src(b, c0), rows1, s1).wait()
                pltpu.async_copy(hv_src(nxt, c0), rows0, s0)
                pltpu.sync_copy(rows1, accz.at[idxb.at[b]], add=True)
                return 0

            lax.fori_loop(0, NACH // 2, sloop, 0)
            pltpu.make_async_copy(hv_src(0, c0), rows0, s0).wait()
            plsc.subcore_barrier()
            pltpu.sync_copy(accz.at[pl.ds(r0, M_PER_TILE)], zrows)

            def dloop(r, _):
                for v in range(CW // 16):
                    zz = zrows[r, pl.ds(v * 16, 16)]
                    cc = crows[r, pl.ds(v * 16, 16)]
                    zrows[r, pl.ds(v * 16, 16)] = zz / jnp.maximum(cc, 1.0)
                return 0

            lax.fori_loop(0, M_PER_TILE, dloop, 0)
            pltpu.sync_copy(zrows,
                            z_hbm.at[pl.ds(r0, M_PER_TILE), pl.ds(c0, CW)])
            plsc.subcore_barrier()

    return k(Hv, batch3, ones_b, z0)


def _tc_init(Vg, Ep, W1v, W1e):
    BM = 512

    def body(vg_ref, e_ref, wv_ref, we_ref, h0_ref, h_ref):
        acc = jnp.dot(vg_ref[...], wv_ref[...], preferred_element_type=jnp.float32)
        acc = acc + jnp.dot(e_ref[...], we_ref[...], preferred_element_type=jnp.float32)
        h0_ref[...] = acc
        h_ref[...] = jnp.maximum(acc, 0.0)

    return pl.pallas_call(
        body,
        grid=(NE_P // BM,),
        in_specs=[
            pl.BlockSpec((BM, 128), lambda i: (i, 0)),
            pl.BlockSpec((BM, 16), lambda i: (i, 0)),
            pl.BlockSpec((128, D_H), lambda i: (0, 0)),
            pl.BlockSpec((16, D_H), lambda i: (0, 0)),
        ],
        out_specs=[
            pl.BlockSpec((BM, D_H), lambda i: (i, 0)),
            pl.BlockSpec((BM, D_H), lambda i: (i, 0)),
        ],
        out_shape=[
            jax.ShapeDtypeStruct((NE_P, D_H), jnp.float32),
            jax.ShapeDtypeStruct((NE_P, D_H), jnp.float32),
        ],
        compiler_params=pltpu.CompilerParams(
            dimension_semantics=("parallel",),
        ),
    )(Vg, Ep, W1v, W1e)


def _pair_swap(x, bm, bk):
    rows = lax.broadcasted_iota(jnp.int32, (bm, bk), 0)
    dn = pltpu.roll(x, bm - 1, 0)
    up = pltpu.roll(x, 1, 0)
    return jnp.where(rows % 2 == 0, dn, up)


def _tc_update(G, H, H0, W_h):
    BM = 512
    BK = 512
    KG = D_H // BK

    def body(g_ref, h_ref, h0_ref, w_ref, out_ref):
        acc = h0_ref[...]
        for k in range(KG):
            sl = pl.ds(k * BK, BK)
            m = (g_ref[:, sl] - _pair_swap(h_ref[:, sl], BM, BK)).astype(jnp.bfloat16)
            acc = acc + jnp.dot(m, w_ref[sl, :], preferred_element_type=jnp.float32)
        out_ref[...] = jnp.maximum(acc, 0.0)

    return pl.pallas_call(
        body,
        grid=(NE_P // BM,),
        in_specs=[
            pl.BlockSpec((BM, D_H), lambda i: (i, 0)),
            pl.BlockSpec((BM, D_H), lambda i: (i, 0)),
            pl.BlockSpec((BM, D_H), lambda i: (i, 0)),
            pl.BlockSpec((D_H, D_H), lambda i: (0, 0)),
        ],
        out_specs=pl.BlockSpec((BM, D_H), lambda i: (i, 0)),
        out_shape=jax.ShapeDtypeStruct((NE_P, D_H), jnp.float32),
        compiler_params=pltpu.CompilerParams(
            dimension_semantics=("parallel",),
        ),
    )(G, H, H0, W_h)


def _tc_final(Vp, Mv, Wov, Wom, bo):
    BM = 512
    BK = 512
    KG = D_H // BK

    def body(v_ref, mv_ref, wv_ref, wm_ref, b_ref, out_ref):
        acc = jnp.dot(v_ref[...], wv_ref[...], preferred_element_type=jnp.float32)
        acc = acc + b_ref[0:1, :]
        for k in range(KG):
            sl = pl.ds(k * BK, BK)
            acc = acc + jnp.dot(mv_ref[:, sl].astype(jnp.bfloat16), wm_ref[sl, :],
                                preferred_element_type=jnp.float32)
        out_ref[...] = jnp.maximum(acc, 0.0)

    return pl.pallas_call(
        body,
        grid=(NA_P // BM,),
        in_specs=[
            pl.BlockSpec((BM, 128), lambda i: (i, 0)),
            pl.BlockSpec((BM, D_H), lambda i: (i, 0)),
            pl.BlockSpec((128, D_H), lambda i: (0, 0)),
            pl.BlockSpec((D_H, D_H), lambda i: (0, 0)),
            pl.BlockSpec((8, D_H), lambda i: (0, 0)),
        ],
        out_specs=pl.BlockSpec((BM, D_H), lambda i: (i, 0)),
        out_shape=jax.ShapeDtypeStruct((NA_P, D_H), jnp.float32),
        compiler_params=pltpu.CompilerParams(
            dimension_semantics=("parallel",),
        ),
    )(Vp, Mv, Wov, Wom, bo)


def kernel(V, E, edge_index, rev_edge_index, batch, W_i, W_h, W_o, b_o):
    del rev_edge_index
    src = edge_index[0]
    dst = edge_index[1]

    Vp = jnp.pad(V, ((0, NA_P - N_ATOMS), (0, 128 - D_V)))
    Ep = jnp.pad(E, ((0, NE_P - N_EDGES), (0, 16 - D_E)))
    src_p = jnp.pad(src, (0, NE_P - N_EDGES))
    dst_p = jnp.pad(dst, (0, NE_P - N_EDGES),
                    constant_values=NA_P - 1)
    batch_p = jnp.pad(batch, (0, NA_P - N_ATOMS),
                      constant_values=NM_P - 1)
    dst3 = dst_p.reshape(NS, NECH, ECH)
    src3 = src_p.reshape(NC * NS, NGCH, GCH)
    src3v = src_p.reshape(NC * NS, 20, 80)
    batch3 = batch_p.reshape(NS, NACH, ACH)
    z0 = jnp.zeros((ZROWS_M, CW), jnp.float32)
    ones_b = jnp.ones((ACH, CW), jnp.float32)

    W1v = jnp.pad(W_i[:D_V], ((0, 128 - D_V), (0, 0)))
    W1e = jnp.pad(W_i[D_V:], ((0, 16 - D_E), (0, 0)))
    Wov = jnp.pad(W_o[:D_V], ((0, 128 - D_V), (0, 0)))
    Wom = W_o[D_V:].astype(jnp.bfloat16)
    bo2 = jnp.broadcast_to(b_o[None, :], (8, D_H))
    W_hb = W_h.astype(jnp.bfloat16)

    Vg = _sc_gather_rows(Vp, src3v)
    H0, H = _tc_init(Vg, Ep, W1v, W1e)
    for _ in range(1, DEPTH):
        mats = _sc_segsum(H, dst3, z0, out16=True)
        G = _sc_gather_matom(mats, src3)
        H = _tc_update(G, H, H0, W_hb)
    Mv = _sc_segsum(H, dst3, z0, out16=False)
    Hv = _tc_final(Vp, Mv, Wov, Wom, bo2)
    Z = _sc_mol_mean(Hv, batch3, ones_b, z0)
    return Z[:N_MOLS]
